# baseline (speedup 1.0000x reference)
import jax
import jax.numpy as jnp
from jax.experimental import pallas as pl
from jax.experimental.pallas import tpu as pltpu

_R = 4096
_C = 1024
_G = 16
_BR = _R // _G


def _body(p_ref, d_ref, m_ref, out_ref, acc_ref):
    i = pl.program_id(0)

    @pl.when(i == 0)
    def _init():
        acc_ref[0] = 0.0
        acc_ref[1] = 0.0

    p = p_ref[...]
    d = d_ref[0]
    m = m_ref[0]
    sel = m > 0
    acc_ref[0] += jnp.sum(jnp.where(sel, jnp.abs(p - d), 0.0))
    acc_ref[1] += jnp.sum(sel.astype(jnp.float32))

    @pl.when(i == pl.num_programs(0) - 1)
    def _fin():
        out_ref[0, 0] = acc_ref[0] / acc_ref[1]


def kernel(pred, gt, PTVs):
    del PTVs
    p2 = pred.reshape(_R, _C)
    g3 = gt.reshape(2, _R, _C)
    out = pl.pallas_call(
        _body,
        grid=(_G,),
        in_specs=[
            pl.BlockSpec((_BR, _C), lambda i: (i, 0)),
            pl.BlockSpec((1, _BR, _C), lambda i: (0, i, 0)),
            pl.BlockSpec((1, _BR, _C), lambda i: (1, i, 0)),
        ],
        out_specs=pl.BlockSpec(memory_space=pltpu.SMEM),
        out_shape=jax.ShapeDtypeStruct((1, 1), jnp.float32),
        scratch_shapes=[pltpu.SMEM((2,), jnp.float32)],
    )(p2, g3, g3)
    return out.reshape(())


# speedup vs baseline: 4.1916x; 4.1916x over previous
import jax
import jax.numpy as jnp
from jax.experimental import pallas as pl
from jax.experimental.pallas import tpu as pltpu

_N = 256
_G = 8
_B = _N // _G


def _body(p_ref, d_ref, m_ref, out_ref, acc_ref):
    i = pl.program_id(0)

    @pl.when(i == 0)
    def _init():
        acc_ref[0] = 0.0
        acc_ref[1] = 0.0

    p = p_ref[...]
    d = d_ref[0]
    m = m_ref[0]
    sel = m > 0
    acc_ref[0] += jnp.sum(jnp.where(sel, jnp.abs(p - d), 0.0))
    acc_ref[1] += jnp.sum(sel.astype(jnp.float32))

    @pl.when(i == pl.num_programs(0) - 1)
    def _fin():
        out_ref[0, 0] = acc_ref[0] / acc_ref[1]


def kernel(pred, gt, PTVs):
    del PTVs
    p3 = pred.reshape(_N, 128, 128)
    g4 = gt.reshape(2, _N, 128, 128)
    out = pl.pallas_call(
        _body,
        grid=(_G,),
        in_specs=[
            pl.BlockSpec((_B, 128, 128), lambda i: (i, 0, 0)),
            pl.BlockSpec((1, _B, 128, 128), lambda i: (0, i, 0, 0)),
            pl.BlockSpec((1, _B, 128, 128), lambda i: (1, i, 0, 0)),
        ],
        out_specs=pl.BlockSpec(memory_space=pltpu.SMEM),
        out_shape=jax.ShapeDtypeStruct((1, 1), jnp.float32),
        scratch_shapes=[pltpu.SMEM((2,), jnp.float32)],
    )(p3, g4, g4)
    return out.reshape(())


# speedup vs baseline: 4.2612x; 1.0166x over previous
import jax
import jax.numpy as jnp
from jax.experimental import pallas as pl
from jax.experimental.pallas import tpu as pltpu

_N = 256
_G = 4
_B = _N // _G


def _body(p_ref, d_ref, m_ref, out_ref, acc_ref):
    i = pl.program_id(0)

    @pl.when(i == 0)
    def _init():
        acc_ref[0] = 0.0
        acc_ref[1] = 0.0

    p = p_ref[...]
    d = d_ref[0]
    m = m_ref[0]
    sel = m > 0
    acc_ref[0] += jnp.sum(jnp.where(sel, jnp.abs(p - d), 0.0))
    acc_ref[1] += jnp.sum(sel.astype(jnp.float32))

    @pl.when(i == pl.num_programs(0) - 1)
    def _fin():
        out_ref[0, 0] = acc_ref[0] / acc_ref[1]


def kernel(pred, gt, PTVs):
    del PTVs
    p3 = pred.reshape(_N, 128, 128)
    g4 = gt.reshape(2, _N, 128, 128)
    out = pl.pallas_call(
        _body,
        grid=(_G,),
        in_specs=[
            pl.BlockSpec((_B, 128, 128), lambda i: (i, 0, 0)),
            pl.BlockSpec((1, _B, 128, 128), lambda i: (0, i, 0, 0)),
            pl.BlockSpec((1, _B, 128, 128), lambda i: (1, i, 0, 0)),
        ],
        out_specs=pl.BlockSpec(memory_space=pltpu.SMEM),
        out_shape=jax.ShapeDtypeStruct((1, 1), jnp.float32),
        scratch_shapes=[pltpu.SMEM((2,), jnp.float32)],
    )(p3, g4, g4)
    return out.reshape(())
